# baseline (speedup 1.0000x reference)
import functools

import jax
import jax.numpy as jnp
from jax import lax
from jax.experimental import pallas as pl
from jax.experimental.pallas import tpu as pltpu
from jax.experimental.pallas import tpu_sc as plsc


def _shift(x, axis, s, forward, identity):
    n = x.shape[axis]
    pad_shape = list(x.shape)
    pad_shape[axis] = s
    pad = jnp.full(pad_shape, identity, dtype=x.dtype)
    if forward:
        body = lax.slice_in_dim(x, 0, n - s, axis=axis)
        return jnp.concatenate([pad, body], axis=axis)
    else:
        body = lax.slice_in_dim(x, s, n, axis=axis)
        return jnp.concatenate([body, pad], axis=axis)


def _cumprod_ks(x, axis, forward):
    n = x.shape[axis]
    s = 1
    while s < n:
        x = x * _shift(x, axis, s, forward, 1.0)
        s *= 2
    return x


def _gate_kernel(xt_ref, wp_ref, dtw_ref, bias_ref, alog_ref,
                 w_ref, f_ref, gg_ref, *, NS, CH, D):
    XT = xt_ref[0]
    wp = wp_ref[...]
    dtw = dtw_ref[...]
    R = dtw.shape[1]
    xdbl = lax.dot_general(XT, wp, (((1,), (1,)), ((), ())),
                           preferred_element_type=jnp.float32)
    dts = lax.dot_general(xdbl[:, 0:R], dtw, (((1,), (1,)), ((), ())),
                          preferred_element_type=jnp.float32)
    sp = jax.nn.softplus(dts + bias_ref[...])
    A = -jnp.exp(alog_ref[...])
    w = jnp.exp(sp * A)
    f = sp * xdbl[:, R:R + 1] * XT
    w_ref[0] = w
    f_ref[0] = f

    wn = _shift(w, 0, 1, False, 0.0)
    W3 = w.reshape(NS, CH, D)
    WN3 = wn.reshape(NS, CH, D)
    F3 = f.reshape(NS, CH, D)

    cps = _cumprod_ks(W3, 1, forward=False)
    sufP = _shift(cps, 1, 1, False, 1.0)
    P2 = cps[:, 0, :]
    E2 = jnp.sum(sufP * F3, axis=1)

    cpp = _cumprod_ks(WN3, 1, forward=True)
    preP = _shift(cpp, 1, 1, True, 1.0)
    Q2 = cpp[:, CH - 1, :]
    S2 = jnp.sum(preP * F3, axis=1)

    g_rows = [jnp.zeros((1, D), jnp.float32)]
    for s in range(1, NS):
        g_rows.append(P2[s - 1:s, :] * g_rows[s - 1] + E2[s - 1:s, :])
    gr_rows = [jnp.zeros((1, D), jnp.float32)] * NS
    for s in range(NS - 2, -1, -1):
        gr_rows[s] = Q2[s + 1:s + 2, :] * gr_rows[s + 1] + S2[s + 1:s + 2, :]
    G2 = jnp.concatenate(g_rows, axis=0).reshape(NS, 1, D)
    GR2 = jnp.concatenate(gr_rows, axis=0).reshape(NS, 1, D)
    gg_ref[0] = jnp.concatenate([G2, GR2], axis=1)


def _sc_scan_kernel(w_hbm, f_hbm, gg_hbm, out_hbm, w_v, f_v, o_v,
                    c_v, *, CH, D, NS):
    NV = D // 16
    c = lax.axis_index("c")
    s = lax.axis_index("s")
    ch = c * (NS // 2) + s

    @pl.when(s < NS // 2)
    def _():
        base = ch * CH
        pltpu.sync_copy(w_hbm.at[pl.ds(base, CH), :],
                        w_v.at[pl.ds(0, CH), :])
        pltpu.sync_copy(f_hbm.at[pl.ds(base, CH), :], f_v)
        pltpu.sync_copy(gg_hbm.at[ch], c_v)

        @pl.when(ch == NS - 1)
        def _():
            for j in range(NV):
                w_v[CH, pl.ds(16 * j, 16)] = jnp.zeros((16,), jnp.float32)

        @pl.when(ch < NS - 1)
        def _():
            pltpu.sync_copy(w_hbm.at[pl.ds(base + CH, 1), :],
                            w_v.at[pl.ds(CH, 1), :])

        def c_fwd(t, H):
            H = list(H)
            for j in range(NV):
                wv = w_v[t, pl.ds(16 * j, 16)]
                fv = f_v[t, pl.ds(16 * j, 16)]
                H[j] = wv * H[j] + fv
                o_v[t, pl.ds(16 * j, 16)] = H[j]
            return tuple(H)

        G = tuple(c_v[0, pl.ds(16 * j, 16)] for j in range(NV))
        lax.fori_loop(0, CH, c_fwd, G)

        def c_bwd(i, H):
            t = CH - 1 - i
            H = list(H)
            for j in range(NV):
                wv = w_v[t + 1, pl.ds(16 * j, 16)]
                fv = f_v[t, pl.ds(16 * j, 16)]
                H[j] = wv * H[j] + fv
                o_v[t, pl.ds(16 * j, 16)] = (
                    o_v[t, pl.ds(16 * j, 16)] + H[j] - fv)
            return tuple(H)

        Gr = tuple(c_v[1, pl.ds(16 * j, 16)] for j in range(NV))
        lax.fori_loop(0, CH, c_bwd, Gr)

        pltpu.sync_copy(o_v, out_hbm.at[pl.ds(base, CH), :])


def _post_kernel(xt_ref, ft_ref, wp_ref, ds_ref, hw_ref, hb_ref, ow_ref,
                 ob_ref, out_ref):
    XT = xt_ref[0]
    FT = ft_ref[0]
    wp = wp_ref[...]
    cw = wp[wp.shape[0] - 1:, :]
    Cs = lax.dot_general(XT, cw, (((1,), (1,)), ((), ())),
                         preferred_element_type=jnp.float32)
    eps = 1e-5
    mu = jnp.mean(FT, axis=-1, keepdims=True)
    var = jnp.mean((FT - mu) ** 2, axis=-1, keepdims=True)
    out = (FT - mu) * lax.rsqrt(var + eps) * hw_ref[...] + hb_ref[...]
    y = out * Cs + ds_ref[...] * XT
    mu2 = jnp.mean(y, axis=-1, keepdims=True)
    var2 = jnp.mean((y - mu2) ** 2, axis=-1, keepdims=True)
    out_ref[0] = (y - mu2) * lax.rsqrt(var2 + eps) * ow_ref[...] + ob_ref[...]


def kernel(x, x_proj_weight, dt_projs_weight, dt_projs_bias, A_logs, Ds,
           h_norm_w, h_norm_b, out_norm_w, out_norm_b):
    B, D, H, W = x.shape
    L = H * W
    NS = 28
    CH = L // NS
    assert CH * NS == L and CH % 8 == 0 and D % 16 == 0

    xt = jnp.transpose(x.reshape(B, D, L), (0, 2, 1)).astype(jnp.float32)
    wp = x_proj_weight[0].astype(jnp.float32)
    dtw = dt_projs_weight[0].astype(jnp.float32)
    bias = dt_projs_bias.reshape(1, D).astype(jnp.float32)
    alog = A_logs.reshape(1, D).astype(jnp.float32)
    ds = Ds.reshape(1, D).astype(jnp.float32)
    hw = h_norm_w.reshape(1, D).astype(jnp.float32)
    hb = h_norm_b.reshape(1, D).astype(jnp.float32)
    ow = out_norm_w.reshape(1, D).astype(jnp.float32)
    ob = out_norm_b.reshape(1, D).astype(jnp.float32)

    vec = pl.BlockSpec((1, D), lambda b: (0, 0))
    mat = lambda shape: pl.BlockSpec(shape, lambda b: (0, 0))
    big = pl.BlockSpec((1, L, D), lambda b: (b, 0, 0))
    car = pl.BlockSpec((1, NS, 2, D), lambda b: (b, 0, 0, 0))
    shp = jax.ShapeDtypeStruct((1, L, D), jnp.float32)
    cshp = jax.ShapeDtypeStruct((1, NS, 2, D), jnp.float32)

    sc_mesh = plsc.VectorSubcoreMesh(core_axis_name="c", subcore_axis_name="s",
                                     num_cores=2, num_subcores=16)
    ys = []
    for b in range(B):
        xtb = xt[b:b + 1]
        w, f, gg = pl.pallas_call(
            functools.partial(_gate_kernel, NS=NS, CH=CH, D=D),
            grid=(1,),
            in_specs=[big, mat(wp.shape), mat(dtw.shape), vec, vec],
            out_specs=[big, big, car],
            out_shape=[shp, shp, cshp],
        )(xtb, wp, dtw, bias, alog)

        ft = pl.kernel(
            functools.partial(_sc_scan_kernel, CH=CH, D=D, NS=NS),
            out_type=jax.ShapeDtypeStruct((L, D), jnp.float32),
            mesh=sc_mesh,
            scratch_types=[
                pltpu.VMEM((CH + 1, D), jnp.float32),
                pltpu.VMEM((CH, D), jnp.float32),
                pltpu.VMEM((CH, D), jnp.float32),
                pltpu.VMEM((2, D), jnp.float32),
            ],
        )(w.reshape(L, D), f.reshape(L, D), gg.reshape(NS, 2, D))

        y = pl.pallas_call(
            _post_kernel,
            grid=(1,),
            in_specs=[big, big, mat(wp.shape), vec, vec, vec, vec, vec],
            out_specs=big,
            out_shape=shp,
        )(xtb, ft.reshape(1, L, D), wp, ds, hw, hb, ow, ob)
        ys.append(y)

    y = jnp.concatenate(ys, axis=0)
    return y.reshape(B, H, W, D).astype(x.dtype)


# speedup vs baseline: 1.0771x; 1.0771x over previous
import functools

import jax
import jax.numpy as jnp
from jax import lax
from jax.experimental import pallas as pl
from jax.experimental.pallas import tpu as pltpu
from jax.experimental.pallas import tpu_sc as plsc


def _shift(x, axis, s, forward, identity):
    n = x.shape[axis]
    pad_shape = list(x.shape)
    pad_shape[axis] = s
    pad = jnp.full(pad_shape, identity, dtype=x.dtype)
    if forward:
        body = lax.slice_in_dim(x, 0, n - s, axis=axis)
        return jnp.concatenate([pad, body], axis=axis)
    else:
        body = lax.slice_in_dim(x, s, n, axis=axis)
        return jnp.concatenate([body, pad], axis=axis)


def _cumprod_ks(x, axis, forward):
    n = x.shape[axis]
    s = 1
    while s < n:
        x = x * _shift(x, axis, s, forward, 1.0)
        s *= 2
    return x


def _gate_kernel(x_ref, wp_ref, dtw_ref, bias_ref, alog_ref,
                 w_ref, f_ref, xt_ref, gg_ref, *, NS, CH, D):
    XT = jnp.transpose(x_ref[0], (1, 0))
    wp = wp_ref[...]
    dtw = dtw_ref[...]
    R = dtw.shape[1]
    xdbl = lax.dot_general(XT, wp, (((1,), (1,)), ((), ())),
                           preferred_element_type=jnp.float32)
    dts = lax.dot_general(xdbl[:, 0:R], dtw, (((1,), (1,)), ((), ())),
                          preferred_element_type=jnp.float32)
    sp = jax.nn.softplus(dts + bias_ref[...])
    A = -jnp.exp(alog_ref[...])
    w = jnp.exp(sp * A)
    f = sp * xdbl[:, R:R + 1] * XT
    w_ref[0] = w
    f_ref[0] = f
    xt_ref[0] = XT

    wn = _shift(w, 0, 1, False, 0.0)
    W3 = w.reshape(NS, CH, D)
    WN3 = wn.reshape(NS, CH, D)
    F3 = f.reshape(NS, CH, D)

    cps = _cumprod_ks(W3, 1, forward=False)
    sufP = _shift(cps, 1, 1, False, 1.0)
    P2 = cps[:, 0, :]
    E2 = jnp.sum(sufP * F3, axis=1)

    cpp = _cumprod_ks(WN3, 1, forward=True)
    preP = _shift(cpp, 1, 1, True, 1.0)
    Q2 = cpp[:, CH - 1, :]
    S2 = jnp.sum(preP * F3, axis=1)

    g_rows = [jnp.zeros((1, D), jnp.float32)]
    for s in range(1, NS):
        g_rows.append(P2[s - 1:s, :] * g_rows[s - 1] + E2[s - 1:s, :])
    gr_rows = [jnp.zeros((1, D), jnp.float32)] * NS
    for s in range(NS - 2, -1, -1):
        gr_rows[s] = Q2[s + 1:s + 2, :] * gr_rows[s + 1] + S2[s + 1:s + 2, :]
    G2 = jnp.concatenate(g_rows, axis=0).reshape(NS, 1, D)
    GR2 = jnp.concatenate(gr_rows, axis=0).reshape(NS, 1, D)
    gg_ref[0] = jnp.concatenate([G2, GR2], axis=1)


def _sc_scan_kernel(w_hbm, f_hbm, gg_hbm, out_hbm, w_v, f_v, o_v,
                    c_v, *, CH, D, NS):
    NV = D // 16
    c = lax.axis_index("c")
    s = lax.axis_index("s")

    @pl.when(s < NS)
    def _():
        base = s * CH
        pltpu.sync_copy(w_hbm.at[c, pl.ds(base, CH), :],
                        w_v.at[pl.ds(0, CH), :])
        pltpu.sync_copy(f_hbm.at[c, pl.ds(base, CH), :], f_v)
        pltpu.sync_copy(gg_hbm.at[c, s], c_v)

        @pl.when(s == NS - 1)
        def _():
            for j in range(NV):
                w_v[CH, pl.ds(16 * j, 16)] = jnp.zeros((16,), jnp.float32)

        @pl.when(s < NS - 1)
        def _():
            pltpu.sync_copy(w_hbm.at[c, pl.ds(base + CH, 1), :],
                            w_v.at[pl.ds(CH, 1), :])

        def c_fwd(t, H):
            H = list(H)
            for j in range(NV):
                wv = w_v[t, pl.ds(16 * j, 16)]
                fv = f_v[t, pl.ds(16 * j, 16)]
                H[j] = wv * H[j] + fv
                o_v[t, pl.ds(16 * j, 16)] = H[j]
            return tuple(H)

        G = tuple(c_v[0, pl.ds(16 * j, 16)] for j in range(NV))
        lax.fori_loop(0, CH, c_fwd, G)

        def c_bwd(i, H):
            t = CH - 1 - i
            H = list(H)
            for j in range(NV):
                wv = w_v[t + 1, pl.ds(16 * j, 16)]
                fv = f_v[t, pl.ds(16 * j, 16)]
                H[j] = wv * H[j] + fv
                o_v[t, pl.ds(16 * j, 16)] = (
                    o_v[t, pl.ds(16 * j, 16)] + H[j] - fv)
            return tuple(H)

        Gr = tuple(c_v[1, pl.ds(16 * j, 16)] for j in range(NV))
        lax.fori_loop(0, CH, c_bwd, Gr)

        pltpu.sync_copy(o_v, out_hbm.at[c, pl.ds(base, CH), :])


def _post_kernel(xt_ref, ft_ref, wp_ref, ds_ref, hw_ref, hb_ref, ow_ref,
                 ob_ref, out_ref):
    XT = xt_ref[0]
    FT = ft_ref[0]
    wp = wp_ref[...]
    cw = wp[wp.shape[0] - 1:, :]
    Cs = lax.dot_general(XT, cw, (((1,), (1,)), ((), ())),
                         preferred_element_type=jnp.float32)
    eps = 1e-5
    mu = jnp.mean(FT, axis=-1, keepdims=True)
    var = jnp.mean((FT - mu) ** 2, axis=-1, keepdims=True)
    out = (FT - mu) * lax.rsqrt(var + eps) * hw_ref[...] + hb_ref[...]
    y = out * Cs + ds_ref[...] * XT
    mu2 = jnp.mean(y, axis=-1, keepdims=True)
    var2 = jnp.mean((y - mu2) ** 2, axis=-1, keepdims=True)
    out_ref[0] = (y - mu2) * lax.rsqrt(var2 + eps) * ow_ref[...] + ob_ref[...]


def kernel(x, x_proj_weight, dt_projs_weight, dt_projs_bias, A_logs, Ds,
           h_norm_w, h_norm_b, out_norm_w, out_norm_b):
    B, D, H, W = x.shape
    L = H * W
    NS = 14
    CH = L // NS
    assert CH * NS == L and CH % 8 == 0 and D % 16 == 0

    x3 = x.reshape(B, D, L).astype(jnp.float32)
    wp = x_proj_weight[0].astype(jnp.float32)
    dtw = dt_projs_weight[0].astype(jnp.float32)
    bias = dt_projs_bias.reshape(1, D).astype(jnp.float32)
    alog = A_logs.reshape(1, D).astype(jnp.float32)
    ds = Ds.reshape(1, D).astype(jnp.float32)
    hw = h_norm_w.reshape(1, D).astype(jnp.float32)
    hb = h_norm_b.reshape(1, D).astype(jnp.float32)
    ow = out_norm_w.reshape(1, D).astype(jnp.float32)
    ob = out_norm_b.reshape(1, D).astype(jnp.float32)

    vec = pl.BlockSpec((1, D), lambda b: (0, 0))
    mat = lambda shape: pl.BlockSpec(shape, lambda b: (0, 0))
    big = pl.BlockSpec((1, L, D), lambda b: (b, 0, 0))
    car = pl.BlockSpec((1, NS, 2, D), lambda b: (b, 0, 0, 0))
    shp = jax.ShapeDtypeStruct((B, L, D), jnp.float32)
    cshp = jax.ShapeDtypeStruct((B, NS, 2, D), jnp.float32)

    w, f, xt, gg = pl.pallas_call(
        functools.partial(_gate_kernel, NS=NS, CH=CH, D=D),
        grid=(B,),
        in_specs=[pl.BlockSpec((1, D, L), lambda b: (b, 0, 0)),
                  mat(wp.shape), mat(dtw.shape), vec, vec],
        out_specs=[big, big, big, car],
        out_shape=[shp, shp, shp, cshp],
    )(x3, wp, dtw, bias, alog)

    sc_mesh = plsc.VectorSubcoreMesh(core_axis_name="c", subcore_axis_name="s",
                                     num_cores=2, num_subcores=16)
    ft = pl.kernel(
        functools.partial(_sc_scan_kernel, CH=CH, D=D, NS=NS),
        out_type=shp,
        mesh=sc_mesh,
        scratch_types=[
            pltpu.VMEM((CH + 1, D), jnp.float32),
            pltpu.VMEM((CH, D), jnp.float32),
            pltpu.VMEM((CH, D), jnp.float32),
            pltpu.VMEM((2, D), jnp.float32),
        ],
    )(w, f, gg)

    y = pl.pallas_call(
        _post_kernel,
        grid=(B,),
        in_specs=[big, big, mat(wp.shape), vec, vec, vec, vec, vec],
        out_specs=big,
        out_shape=shp,
    )(xt, ft, wp, ds, hw, hb, ow, ob)

    return y.reshape(B, H, W, D).astype(x.dtype)


# speedup vs baseline: 1.1814x; 1.0968x over previous
import functools

import jax
import jax.numpy as jnp
from jax import lax
from jax.experimental import pallas as pl
from jax.experimental.pallas import tpu as pltpu
from jax.experimental.pallas import tpu_sc as plsc


def _shift(x, axis, s, forward, identity):
    n = x.shape[axis]
    pad_shape = list(x.shape)
    pad_shape[axis] = s
    pad = jnp.full(pad_shape, identity, dtype=x.dtype)
    if forward:
        body = lax.slice_in_dim(x, 0, n - s, axis=axis)
        return jnp.concatenate([pad, body], axis=axis)
    else:
        body = lax.slice_in_dim(x, s, n, axis=axis)
        return jnp.concatenate([body, pad], axis=axis)


def _cumprod_ks(x, axis, forward):
    n = x.shape[axis]
    s = 1
    while s < n:
        x = x * _shift(x, axis, s, forward, 1.0)
        s *= 2
    return x


def _gate_kernel(xt_ref, wp_ref, dtw_ref, bias_ref, alog_ref,
                 w_ref, f_ref, gg_ref, *, NS, CH, D):
    XT = xt_ref[0]
    wp = wp_ref[...]
    dtw = dtw_ref[...]
    R = dtw.shape[1]
    xdbl = lax.dot_general(XT, wp, (((1,), (1,)), ((), ())),
                           preferred_element_type=jnp.float32)
    dts = lax.dot_general(xdbl[:, 0:R], dtw, (((1,), (1,)), ((), ())),
                          preferred_element_type=jnp.float32)
    sp = jax.nn.softplus(dts + bias_ref[...])
    A = -jnp.exp(alog_ref[...])
    w = jnp.exp(sp * A)
    f = sp * xdbl[:, R:R + 1] * XT
    w_ref[0] = w
    f_ref[0] = f

    wn = _shift(w, 0, 1, False, 0.0)
    W3 = w.reshape(NS, CH, D)
    WN3 = wn.reshape(NS, CH, D)
    F3 = f.reshape(NS, CH, D)

    cps = _cumprod_ks(W3, 1, forward=False)
    sufP = _shift(cps, 1, 1, False, 1.0)
    P2 = cps[:, 0, :]
    E2 = jnp.sum(sufP * F3, axis=1)

    cpp = _cumprod_ks(WN3, 1, forward=True)
    preP = _shift(cpp, 1, 1, True, 1.0)
    Q2 = cpp[:, CH - 1, :]
    S2 = jnp.sum(preP * F3, axis=1)

    g_rows = [jnp.zeros((1, D), jnp.float32)]
    for s in range(1, NS):
        g_rows.append(P2[s - 1:s, :] * g_rows[s - 1] + E2[s - 1:s, :])
    gr_rows = [jnp.zeros((1, D), jnp.float32)] * NS
    for s in range(NS - 2, -1, -1):
        gr_rows[s] = Q2[s + 1:s + 2, :] * gr_rows[s + 1] + S2[s + 1:s + 2, :]
    G2 = jnp.concatenate(g_rows, axis=0).reshape(NS, 1, D)
    GR2 = jnp.concatenate(gr_rows, axis=0).reshape(NS, 1, D)
    gg_ref[0] = jnp.concatenate([G2, GR2], axis=1)


def _sc_scan_kernel(w_hbm, f_hbm, gg_hbm, out_hbm, w_v, f_v, o_v,
                    c_v, *, CH, D, NS):
    NV = D // 16
    c = lax.axis_index("c")
    s = lax.axis_index("s")

    @pl.when(s < NS)
    def _():
        base = s * CH
        pltpu.sync_copy(w_hbm.at[c, pl.ds(base, CH), :],
                        w_v.at[pl.ds(0, CH), :])
        pltpu.sync_copy(f_hbm.at[c, pl.ds(base, CH), :], f_v)
        pltpu.sync_copy(gg_hbm.at[c, s], c_v)

        @pl.when(s == NS - 1)
        def _():
            for j in range(NV):
                w_v[CH, pl.ds(16 * j, 16)] = jnp.zeros((16,), jnp.float32)

        @pl.when(s < NS - 1)
        def _():
            pltpu.sync_copy(w_hbm.at[c, pl.ds(base + CH, 1), :],
                            w_v.at[pl.ds(CH, 1), :])

        def c_fwd(t, H):
            H = list(H)
            for j in range(NV):
                wv = w_v[t, pl.ds(16 * j, 16)]
                fv = f_v[t, pl.ds(16 * j, 16)]
                H[j] = wv * H[j] + fv
                o_v[t, pl.ds(16 * j, 16)] = H[j]
            return tuple(H)

        G = tuple(c_v[0, pl.ds(16 * j, 16)] for j in range(NV))
        lax.fori_loop(0, CH, c_fwd, G)

        def c_bwd(i, H):
            t = CH - 1 - i
            H = list(H)
            for j in range(NV):
                wv = w_v[t + 1, pl.ds(16 * j, 16)]
                fv = f_v[t, pl.ds(16 * j, 16)]
                H[j] = wv * H[j] + fv
                o_v[t, pl.ds(16 * j, 16)] = (
                    o_v[t, pl.ds(16 * j, 16)] + H[j] - fv)
            return tuple(H)

        Gr = tuple(c_v[1, pl.ds(16 * j, 16)] for j in range(NV))
        lax.fori_loop(0, CH, c_bwd, Gr)

        pltpu.sync_copy(o_v, out_hbm.at[c, pl.ds(base, CH), :])


def _post_kernel(xt_ref, ft_ref, wp_ref, ds_ref, hw_ref, hb_ref, ow_ref,
                 ob_ref, out_ref):
    XT = xt_ref[0]
    FT = ft_ref[0]
    wp = wp_ref[...]
    cw = wp[wp.shape[0] - 1:, :]
    Cs = lax.dot_general(XT, cw, (((1,), (1,)), ((), ())),
                         preferred_element_type=jnp.float32)
    eps = 1e-5
    mu = jnp.mean(FT, axis=-1, keepdims=True)
    var = jnp.mean((FT - mu) ** 2, axis=-1, keepdims=True)
    out = (FT - mu) * lax.rsqrt(var + eps) * hw_ref[...] + hb_ref[...]
    y = out * Cs + ds_ref[...] * XT
    mu2 = jnp.mean(y, axis=-1, keepdims=True)
    var2 = jnp.mean((y - mu2) ** 2, axis=-1, keepdims=True)
    out_ref[0] = (y - mu2) * lax.rsqrt(var2 + eps) * ow_ref[...] + ob_ref[...]


def kernel(x, x_proj_weight, dt_projs_weight, dt_projs_bias, A_logs, Ds,
           h_norm_w, h_norm_b, out_norm_w, out_norm_b):
    B, D, H, W = x.shape
    L = H * W
    NS = 14
    CH = L // NS
    assert CH * NS == L and CH % 8 == 0 and D % 16 == 0

    xt = jnp.transpose(x.reshape(B, D, L), (0, 2, 1)).astype(jnp.float32)
    wp = x_proj_weight[0].astype(jnp.float32)
    dtw = dt_projs_weight[0].astype(jnp.float32)
    bias = dt_projs_bias.reshape(1, D).astype(jnp.float32)
    alog = A_logs.reshape(1, D).astype(jnp.float32)
    ds = Ds.reshape(1, D).astype(jnp.float32)
    hw = h_norm_w.reshape(1, D).astype(jnp.float32)
    hb = h_norm_b.reshape(1, D).astype(jnp.float32)
    ow = out_norm_w.reshape(1, D).astype(jnp.float32)
    ob = out_norm_b.reshape(1, D).astype(jnp.float32)

    vec = pl.BlockSpec((1, D), lambda b: (0, 0))
    mat = lambda shape: pl.BlockSpec(shape, lambda b: (0, 0))
    big = pl.BlockSpec((1, L, D), lambda b: (b, 0, 0))
    car = pl.BlockSpec((1, NS, 2, D), lambda b: (b, 0, 0, 0))
    shp = jax.ShapeDtypeStruct((B, L, D), jnp.float32)
    cshp = jax.ShapeDtypeStruct((B, NS, 2, D), jnp.float32)

    w, f, gg = pl.pallas_call(
        functools.partial(_gate_kernel, NS=NS, CH=CH, D=D),
        grid=(B,),
        in_specs=[big, mat(wp.shape), mat(dtw.shape), vec, vec],
        out_specs=[big, big, car],
        out_shape=[shp, shp, cshp],
    )(xt, wp, dtw, bias, alog)

    sc_mesh = plsc.VectorSubcoreMesh(core_axis_name="c", subcore_axis_name="s",
                                     num_cores=2, num_subcores=16)
    ft = pl.kernel(
        functools.partial(_sc_scan_kernel, CH=CH, D=D, NS=NS),
        out_type=shp,
        mesh=sc_mesh,
        scratch_types=[
            pltpu.VMEM((CH + 1, D), jnp.float32),
            pltpu.VMEM((CH, D), jnp.float32),
            pltpu.VMEM((CH, D), jnp.float32),
            pltpu.VMEM((2, D), jnp.float32),
        ],
    )(w, f, gg)

    y = pl.pallas_call(
        _post_kernel,
        grid=(B,),
        in_specs=[big, big, mat(wp.shape), vec, vec, vec, vec, vec],
        out_specs=big,
        out_shape=shp,
    )(xt, ft, wp, ds, hw, hb, ow, ob)

    return y.reshape(B, H, W, D).astype(x.dtype)


# speedup vs baseline: 1.2252x; 1.0371x over previous
import functools

import jax
import jax.numpy as jnp
from jax import lax
from jax.experimental import pallas as pl
from jax.experimental.pallas import tpu as pltpu
from jax.experimental.pallas import tpu_sc as plsc


def _shift(x, axis, s, forward, identity):
    n = x.shape[axis]
    pad_shape = list(x.shape)
    pad_shape[axis] = s
    pad = jnp.full(pad_shape, identity, dtype=x.dtype)
    if forward:
        body = lax.slice_in_dim(x, 0, n - s, axis=axis)
        return jnp.concatenate([pad, body], axis=axis)
    else:
        body = lax.slice_in_dim(x, s, n, axis=axis)
        return jnp.concatenate([body, pad], axis=axis)


def _cumprod_ks(x, axis, forward):
    n = x.shape[axis]
    s = 1
    while s < n:
        x = x * _shift(x, axis, s, forward, 1.0)
        s *= 2
    return x


def _gate_kernel(xt_ref, wp_ref, dtw_ref, bias_ref, alog_ref,
                 w_ref, f_ref, gg_ref, *, NS, CH, D):
    XT = xt_ref[0]
    wp = wp_ref[...]
    dtw = dtw_ref[...]
    R = dtw.shape[1]
    xdbl = lax.dot_general(XT, wp, (((1,), (1,)), ((), ())),
                           preferred_element_type=jnp.float32)
    dts = lax.dot_general(xdbl[:, 0:R], dtw, (((1,), (1,)), ((), ())),
                          preferred_element_type=jnp.float32)
    sp = jax.nn.softplus(dts + bias_ref[...])
    A = -jnp.exp(alog_ref[...])
    w = jnp.exp(sp * A)
    f = sp * xdbl[:, R:R + 1] * XT
    w_ref[0] = w
    f_ref[0] = f

    wn = _shift(w, 0, 1, False, 0.0)
    W3 = w.reshape(NS, CH, D)
    WN3 = wn.reshape(NS, CH, D)
    F3 = f.reshape(NS, CH, D)

    cps = _cumprod_ks(W3, 1, forward=False)
    sufP = _shift(cps, 1, 1, False, 1.0)
    P2 = cps[:, 0, :]
    E2 = jnp.sum(sufP * F3, axis=1)

    cpp = _cumprod_ks(WN3, 1, forward=True)
    preP = _shift(cpp, 1, 1, True, 1.0)
    Q2 = cpp[:, CH - 1, :]
    S2 = jnp.sum(preP * F3, axis=1)

    g_rows = [jnp.zeros((1, D), jnp.float32)]
    for s in range(1, NS):
        g_rows.append(P2[s - 1:s, :] * g_rows[s - 1] + E2[s - 1:s, :])
    gr_rows = [jnp.zeros((1, D), jnp.float32)] * NS
    for s in range(NS - 2, -1, -1):
        gr_rows[s] = Q2[s + 1:s + 2, :] * gr_rows[s + 1] + S2[s + 1:s + 2, :]
    G2 = jnp.concatenate(g_rows, axis=0).reshape(NS, 1, D)
    GR2 = jnp.concatenate(gr_rows, axis=0).reshape(NS, 1, D)
    gg_ref[0] = jnp.concatenate([G2, GR2], axis=1)


def _sc_scan_kernel(w_hbm, f_hbm, gg_hbm, out_hbm, w_v, f_v, o_v,
                    c_v, sem, *, CH, D, NS):
    NV = D // 16
    c = lax.axis_index("c")
    s = lax.axis_index("s")

    @pl.when(s < NS)
    def _():
        base = s * CH
        cp_w = pltpu.async_copy(w_hbm.at[c, pl.ds(base, CH), :],
                                w_v.at[pl.ds(0, CH), :], sem)
        cp_f = pltpu.async_copy(f_hbm.at[c, pl.ds(base, CH), :], f_v, sem)
        cp_g = pltpu.async_copy(gg_hbm.at[c, s], c_v, sem)

        @pl.when(s == NS - 1)
        def _():
            for j in range(NV):
                w_v[CH, pl.ds(16 * j, 16)] = jnp.zeros((16,), jnp.float32)

        @pl.when(s < NS - 1)
        def _():
            pltpu.async_copy(w_hbm.at[c, pl.ds(base + CH, 1), :],
                             w_v.at[pl.ds(CH, 1), :], sem).wait()

        cp_w.wait()
        cp_f.wait()
        cp_g.wait()

        def c_fwd(t, H):
            H = list(H)
            for j in range(NV):
                wv = w_v[t, pl.ds(16 * j, 16)]
                fv = f_v[t, pl.ds(16 * j, 16)]
                H[j] = wv * H[j] + fv
                o_v[t, pl.ds(16 * j, 16)] = H[j]
            return tuple(H)

        G = tuple(c_v[0, pl.ds(16 * j, 16)] for j in range(NV))
        lax.fori_loop(0, CH, c_fwd, G)

        def c_bwd(i, H):
            t = CH - 1 - i
            H = list(H)
            for j in range(NV):
                wv = w_v[t + 1, pl.ds(16 * j, 16)]
                fv = f_v[t, pl.ds(16 * j, 16)]
                H[j] = wv * H[j] + fv
                o_v[t, pl.ds(16 * j, 16)] = (
                    o_v[t, pl.ds(16 * j, 16)] + H[j] - fv)
            return tuple(H)

        Gr = tuple(c_v[1, pl.ds(16 * j, 16)] for j in range(NV))
        lax.fori_loop(0, CH, c_bwd, Gr)

        pltpu.sync_copy(o_v, out_hbm.at[c, pl.ds(base, CH), :])


def _post_kernel(xt_ref, ft_ref, wp_ref, ds_ref, hw_ref, hb_ref, ow_ref,
                 ob_ref, out_ref):
    XT = xt_ref[0]
    FT = ft_ref[0]
    wp = wp_ref[...]
    cw = wp[wp.shape[0] - 1:, :]
    Cs = lax.dot_general(XT, cw, (((1,), (1,)), ((), ())),
                         preferred_element_type=jnp.float32)
    eps = 1e-5
    mu = jnp.mean(FT, axis=-1, keepdims=True)
    var = jnp.mean((FT - mu) ** 2, axis=-1, keepdims=True)
    out = (FT - mu) * lax.rsqrt(var + eps) * hw_ref[...] + hb_ref[...]
    y = out * Cs + ds_ref[...] * XT
    mu2 = jnp.mean(y, axis=-1, keepdims=True)
    var2 = jnp.mean((y - mu2) ** 2, axis=-1, keepdims=True)
    out_ref[0] = (y - mu2) * lax.rsqrt(var2 + eps) * ow_ref[...] + ob_ref[...]


def kernel(x, x_proj_weight, dt_projs_weight, dt_projs_bias, A_logs, Ds,
           h_norm_w, h_norm_b, out_norm_w, out_norm_b):
    B, D, H, W = x.shape
    L = H * W
    NS = 14
    CH = L // NS
    assert CH * NS == L and CH % 8 == 0 and D % 16 == 0

    xt = jnp.transpose(x.reshape(B, D, L), (0, 2, 1)).astype(jnp.float32)
    wp = x_proj_weight[0].astype(jnp.float32)
    dtw = dt_projs_weight[0].astype(jnp.float32)
    bias = dt_projs_bias.reshape(1, D).astype(jnp.float32)
    alog = A_logs.reshape(1, D).astype(jnp.float32)
    ds = Ds.reshape(1, D).astype(jnp.float32)
    hw = h_norm_w.reshape(1, D).astype(jnp.float32)
    hb = h_norm_b.reshape(1, D).astype(jnp.float32)
    ow = out_norm_w.reshape(1, D).astype(jnp.float32)
    ob = out_norm_b.reshape(1, D).astype(jnp.float32)

    vec = pl.BlockSpec((1, D), lambda b: (0, 0))
    mat = lambda shape: pl.BlockSpec(shape, lambda b: (0, 0))
    big = pl.BlockSpec((1, L, D), lambda b: (b, 0, 0))
    car = pl.BlockSpec((1, NS, 2, D), lambda b: (b, 0, 0, 0))
    shp = jax.ShapeDtypeStruct((B, L, D), jnp.float32)
    cshp = jax.ShapeDtypeStruct((B, NS, 2, D), jnp.float32)

    w, f, gg = pl.pallas_call(
        functools.partial(_gate_kernel, NS=NS, CH=CH, D=D),
        grid=(B,),
        in_specs=[big, mat(wp.shape), mat(dtw.shape), vec, vec],
        out_specs=[big, big, car],
        out_shape=[shp, shp, cshp],
    )(xt, wp, dtw, bias, alog)

    sc_mesh = plsc.VectorSubcoreMesh(core_axis_name="c", subcore_axis_name="s",
                                     num_cores=2, num_subcores=16)
    ft = pl.kernel(
        functools.partial(_sc_scan_kernel, CH=CH, D=D, NS=NS),
        out_type=shp,
        mesh=sc_mesh,
        scratch_types=[
            pltpu.VMEM((CH + 1, D), jnp.float32),
            pltpu.VMEM((CH, D), jnp.float32),
            pltpu.VMEM((CH, D), jnp.float32),
            pltpu.VMEM((2, D), jnp.float32),
            pltpu.SemaphoreType.DMA,
        ],
    )(w, f, gg)

    y = pl.pallas_call(
        _post_kernel,
        grid=(B,),
        in_specs=[big, big, mat(wp.shape), vec, vec, vec, vec, vec],
        out_specs=big,
        out_shape=shp,
    )(xt, ft, wp, ds, hw, hb, ow, ob)

    return y.reshape(B, H, W, D).astype(x.dtype)
